# baseline (speedup 1.0000x reference)
import functools

import jax
import jax.numpy as jnp
from jax import lax
from jax.experimental import pallas as pl
from jax.experimental.pallas import tpu as pltpu
from jax.experimental.pallas import tpu_sc as plsc

N_NODES = 10000
N_PAD = 10240
E_EDGES = 320000
D = 128

NC = 2
NS = 16
NW = NC * NS
E_PER_W = 10240
E_PAD = E_PER_W * NW
CHUNK = 128
NCHUNKS = E_PER_W // CHUNK
ROWS_PER_TILE = N_PAD // NS

NCH_PAIR = 2 * NCHUNKS
NCH_C0 = 100
NCH_C1 = NCH_PAIR - NCH_C0

_MESH = plsc.VectorSubcoreMesh(core_axis_name="c", subcore_axis_name="s")


def _fill_f32(ref, rows, width, val):
    groups = width // 16
    v = jnp.full((16,), val, jnp.float32)

    def body(i, _):
        r = i // groups
        g = i % groups
        ref[r, pl.ds(g * 16, 16)] = v
        return 0

    lax.fori_loop(0, rows * groups, body, 0)


@functools.partial(
    pl.kernel,
    out_type=jax.ShapeDtypeStruct((NC * N_PAD, D), jnp.float32),
    mesh=_MESH,
    scratch_types=[
        pltpu.VMEM((2, CHUNK), jnp.int32),
        pltpu.VMEM((CHUNK,), jnp.int32),
        pltpu.VMEM((2, CHUNK, D), jnp.float32),
        pltpu.VMEM_SHARED((N_PAD, D), jnp.float32),
        pltpu.SemaphoreType.DMA,
        pltpu.SemaphoreType.DMA,
    ],
)
def _spmm_kernel(h_hbm, src_hbm, dst_hbm, agg_hbm,
                 sidx_v, didx_v, rows_v, sh_agg, sem0, sem1):
    cid = lax.axis_index("c")
    sid = lax.axis_index("s")
    nch = jnp.where(cid == 0, NCH_C0, NCH_C1)
    base = (sid * NCH_PAIR + jnp.where(cid == 0, 0, NCH_C0)) * CHUNK
    sems = (sem0, sem1)

    _fill_f32(rows_v.at[0], CHUNK, D, 0.0)
    for k in range(ROWS_PER_TILE // CHUNK):
        row0 = sid * ROWS_PER_TILE + k * CHUNK
        pltpu.sync_copy(rows_v.at[0], sh_agg.at[pl.ds(row0, CHUNK)])
    plsc.subcore_barrier()

    pltpu.sync_copy(src_hbm.at[pl.ds(base, CHUNK)], sidx_v.at[0])
    pltpu.async_copy(h_hbm.at[sidx_v.at[0]], rows_v.at[0], sem0)

    def chunk(ci, _):
        b = lax.rem(ci, 2)
        nb = lax.rem(ci + 1, 2)

        @pl.when(ci + 1 < nch)
        def _prefetch():
            off = base + (ci + 1) * CHUNK
            pltpu.sync_copy(src_hbm.at[pl.ds(off, CHUNK)], sidx_v.at[nb])
            for i, s in enumerate(sems):
                @pl.when(nb == i)
                def _go():
                    pltpu.async_copy(h_hbm.at[sidx_v.at[nb]], rows_v.at[nb], s)

        for i, s in enumerate(sems):
            @pl.when(b == i)
            def _wait():
                pltpu.make_async_copy(h_hbm.at[sidx_v.at[b]], rows_v.at[b],
                                      s).wait()
        pltpu.sync_copy(dst_hbm.at[pl.ds(base + ci * CHUNK, CHUNK)], didx_v)
        pltpu.sync_copy(rows_v.at[b], sh_agg.at[didx_v], add=True)
        return 0

    lax.fori_loop(0, nch, chunk, 0)
    plsc.subcore_barrier()

    row0 = sid * ROWS_PER_TILE
    out0 = cid * N_PAD + sid * ROWS_PER_TILE
    pltpu.sync_copy(sh_agg.at[pl.ds(row0, ROWS_PER_TILE)],
                    agg_hbm.at[pl.ds(out0, ROWS_PER_TILE)])


@functools.partial(
    pl.kernel,
    out_type=jax.ShapeDtypeStruct((NC * N_PAD, D), jnp.float32),
    mesh=_MESH,
    scratch_types=[
        pltpu.VMEM((CHUNK,), jnp.int32),
        pltpu.VMEM((CHUNK, D), jnp.float32),
        pltpu.VMEM_SHARED((N_PAD, D), jnp.float32),
    ],
)
def _deg_kernel(idx_hbm, deg_hbm, idx_v, ones_v, sh_deg):
    cid = lax.axis_index("c")
    sid = lax.axis_index("s")
    wid = sid * NC + cid
    base = wid * E_PER_W

    _fill_f32(ones_v, CHUNK, D, 0.0)
    for k in range(ROWS_PER_TILE // CHUNK):
        row0 = sid * ROWS_PER_TILE + k * CHUNK
        pltpu.sync_copy(ones_v, sh_deg.at[pl.ds(row0, CHUNK)])
    _fill_f32(ones_v, CHUNK, D, 1.0)
    plsc.subcore_barrier()

    def chunk(ci, _):
        pltpu.sync_copy(idx_hbm.at[pl.ds(base + ci * CHUNK, CHUNK)], idx_v)
        pltpu.sync_copy(ones_v, sh_deg.at[idx_v], add=True)
        return 0

    lax.fori_loop(0, NCHUNKS, chunk, 0)
    plsc.subcore_barrier()

    row0 = sid * ROWS_PER_TILE
    out0 = cid * N_PAD + sid * ROWS_PER_TILE
    pltpu.sync_copy(sh_deg.at[pl.ds(row0, ROWS_PER_TILE)],
                    deg_hbm.at[pl.ds(out0, ROWS_PER_TILE)])


BLK = 1024
GRID = N_PAD // BLK


def _norm_from_deg(d0, d1):
    deg = (d0 + d1)[:, 0:1]
    return jnp.where(deg > 0, lax.rsqrt(jnp.maximum(deg, 1.0)), 0.0)


def _scale_body(x_ref, dego0_ref, dego1_ref, o_ref):
    norm_out = _norm_from_deg(dego0_ref[...], dego1_ref[...])
    o_ref[...] = x_ref[...] * norm_out


def _scale_call(x_pad, dego_p):
    return pl.pallas_call(
        _scale_body,
        out_shape=jax.ShapeDtypeStruct((N_PAD, D), jnp.float32),
        grid=(GRID,),
        in_specs=[
            pl.BlockSpec((BLK, D), lambda i: (i, 0)),
            pl.BlockSpec((BLK, D), lambda i: (i, 0)),
            pl.BlockSpec((BLK, D), lambda i: (i + GRID, 0)),
        ],
        out_specs=pl.BlockSpec((BLK, D), lambda i: (i, 0)),
    )(x_pad, dego_p, dego_p)


def _mm_body(a0_ref, a1_ref, degi0_ref, degi1_ref, dego0_ref, dego1_ref,
             w_ref, b_ref, o_ref, *, relu_and_scale):
    norm_in = _norm_from_deg(degi0_ref[...], degi1_ref[...])
    agg = (a0_ref[...] + a1_ref[...]) * norm_in
    out = jnp.dot(agg, w_ref[...], preferred_element_type=jnp.float32)
    out = out + b_ref[...]
    if relu_and_scale:
        out = jnp.maximum(out, 0.0)
        out = out * _norm_from_deg(dego0_ref[...], dego1_ref[...])
    o_ref[...] = out


def _mm_call(agg_p, degi_p, dego_p, w, b, relu_and_scale):
    body = functools.partial(_mm_body, relu_and_scale=relu_and_scale)
    return pl.pallas_call(
        body,
        out_shape=jax.ShapeDtypeStruct((N_PAD, D), jnp.float32),
        grid=(GRID,),
        in_specs=[
            pl.BlockSpec((BLK, D), lambda i: (i, 0)),
            pl.BlockSpec((BLK, D), lambda i: (i + GRID, 0)),
            pl.BlockSpec((BLK, D), lambda i: (i, 0)),
            pl.BlockSpec((BLK, D), lambda i: (i + GRID, 0)),
            pl.BlockSpec((BLK, D), lambda i: (i, 0)),
            pl.BlockSpec((BLK, D), lambda i: (i + GRID, 0)),
            pl.BlockSpec((D, D), lambda i: (0, 0)),
            pl.BlockSpec((1, D), lambda i: (0, 0)),
        ],
        out_specs=pl.BlockSpec((BLK, D), lambda i: (i, 0)),
    )(agg_p, agg_p, degi_p, degi_p, dego_p, dego_p, w, b)


def kernel(x, edge_index, W1, b1, W2, b2):
    src = edge_index[0]
    dst = edge_index[1]
    pad_idx = jnp.full((E_PAD - E_EDGES,), N_NODES, jnp.int32)
    src_p = jnp.concatenate([src, pad_idx])
    dst_p = jnp.concatenate([dst, pad_idx])
    x_pad = jnp.pad(x, ((0, N_PAD - N_NODES), (0, 0)))
    b1r = b1.reshape(1, D)
    b2r = b2.reshape(1, D)

    dego_p = _deg_kernel(src_p)
    degi_p = _deg_kernel(dst_p)
    h1 = _scale_call(x_pad, dego_p)
    agg1 = _spmm_kernel(h1, src_p, dst_p)
    h2 = _mm_call(agg1, degi_p, dego_p, W1, b1r, True)
    agg2 = _spmm_kernel(h2, src_p, dst_p)
    out = _mm_call(agg2, degi_p, dego_p, W2, b2r, False)
    return out[:N_NODES]


# speedup vs baseline: 1.0094x; 1.0094x over previous
import functools

import jax
import jax.numpy as jnp
from jax import lax
from jax.experimental import pallas as pl
from jax.experimental.pallas import tpu as pltpu
from jax.experimental.pallas import tpu_sc as plsc

N_NODES = 10000
N_PAD = 10240
E_EDGES = 320000
D = 128

NC = 2
NS = 16
NW = NC * NS
E_PER_W = 10240
E_PAD = E_PER_W * NW
CHUNK = 128
NCHUNKS = E_PER_W // CHUNK
ROWS_PER_TILE = N_PAD // NS

NCH_PAIR = 2 * NCHUNKS
NCH_C0 = 114
NCH_C1 = NCH_PAIR - NCH_C0

_MESH = plsc.VectorSubcoreMesh(core_axis_name="c", subcore_axis_name="s")


def _fill_f32(ref, rows, width, val):
    groups = width // 16
    v = jnp.full((16,), val, jnp.float32)

    def body(i, _):
        r = i // groups
        g = i % groups
        ref[r, pl.ds(g * 16, 16)] = v
        return 0

    lax.fori_loop(0, rows * groups, body, 0)


@functools.partial(
    pl.kernel,
    out_type=jax.ShapeDtypeStruct((NC * N_PAD, D), jnp.float32),
    mesh=_MESH,
    scratch_types=[
        pltpu.VMEM((2, CHUNK), jnp.int32),
        pltpu.VMEM((CHUNK,), jnp.int32),
        pltpu.VMEM((2, CHUNK, D), jnp.float32),
        pltpu.VMEM_SHARED((N_PAD, D), jnp.float32),
        pltpu.SemaphoreType.DMA,
        pltpu.SemaphoreType.DMA,
    ],
)
def _spmm_kernel(h_hbm, src_hbm, dst_hbm, agg_hbm,
                 sidx_v, didx_v, rows_v, sh_agg, sem0, sem1):
    cid = lax.axis_index("c")
    sid = lax.axis_index("s")
    nch = jnp.where(cid == 0, NCH_C0, NCH_C1)
    base = (sid * NCH_PAIR + jnp.where(cid == 0, 0, NCH_C0)) * CHUNK
    sems = (sem0, sem1)

    _fill_f32(rows_v.at[0], CHUNK, D, 0.0)
    for k in range(ROWS_PER_TILE // CHUNK):
        row0 = sid * ROWS_PER_TILE + k * CHUNK
        pltpu.sync_copy(rows_v.at[0], sh_agg.at[pl.ds(row0, CHUNK)])
    plsc.subcore_barrier()

    pltpu.sync_copy(src_hbm.at[pl.ds(base, CHUNK)], sidx_v.at[0])
    pltpu.async_copy(h_hbm.at[sidx_v.at[0]], rows_v.at[0], sem0)

    def chunk(ci, _):
        b = lax.rem(ci, 2)
        nb = lax.rem(ci + 1, 2)

        @pl.when(ci + 1 < nch)
        def _prefetch():
            off = base + (ci + 1) * CHUNK
            pltpu.sync_copy(src_hbm.at[pl.ds(off, CHUNK)], sidx_v.at[nb])
            for i, s in enumerate(sems):
                @pl.when(nb == i)
                def _go():
                    pltpu.async_copy(h_hbm.at[sidx_v.at[nb]], rows_v.at[nb], s)

        for i, s in enumerate(sems):
            @pl.when(b == i)
            def _wait():
                pltpu.make_async_copy(h_hbm.at[sidx_v.at[b]], rows_v.at[b],
                                      s).wait()
        pltpu.sync_copy(dst_hbm.at[pl.ds(base + ci * CHUNK, CHUNK)], didx_v)
        pltpu.sync_copy(rows_v.at[b], sh_agg.at[didx_v], add=True)
        return 0

    lax.fori_loop(0, nch, chunk, 0)
    plsc.subcore_barrier()

    row0 = sid * ROWS_PER_TILE
    out0 = cid * N_PAD + sid * ROWS_PER_TILE
    pltpu.sync_copy(sh_agg.at[pl.ds(row0, ROWS_PER_TILE)],
                    agg_hbm.at[pl.ds(out0, ROWS_PER_TILE)])


@functools.partial(
    pl.kernel,
    out_type=jax.ShapeDtypeStruct((NC * N_PAD, D), jnp.float32),
    mesh=_MESH,
    scratch_types=[
        pltpu.VMEM((CHUNK,), jnp.int32),
        pltpu.VMEM((CHUNK, D), jnp.float32),
        pltpu.VMEM_SHARED((N_PAD, D), jnp.float32),
    ],
)
def _deg_kernel(idx_hbm, deg_hbm, idx_v, ones_v, sh_deg):
    cid = lax.axis_index("c")
    sid = lax.axis_index("s")
    wid = sid * NC + cid
    base = wid * E_PER_W

    _fill_f32(ones_v, CHUNK, D, 0.0)
    for k in range(ROWS_PER_TILE // CHUNK):
        row0 = sid * ROWS_PER_TILE + k * CHUNK
        pltpu.sync_copy(ones_v, sh_deg.at[pl.ds(row0, CHUNK)])
    _fill_f32(ones_v, CHUNK, D, 1.0)
    plsc.subcore_barrier()

    def chunk(ci, _):
        pltpu.sync_copy(idx_hbm.at[pl.ds(base + ci * CHUNK, CHUNK)], idx_v)
        pltpu.sync_copy(ones_v, sh_deg.at[idx_v], add=True)
        return 0

    lax.fori_loop(0, NCHUNKS, chunk, 0)
    plsc.subcore_barrier()

    row0 = sid * ROWS_PER_TILE
    out0 = cid * N_PAD + sid * ROWS_PER_TILE
    pltpu.sync_copy(sh_deg.at[pl.ds(row0, ROWS_PER_TILE)],
                    deg_hbm.at[pl.ds(out0, ROWS_PER_TILE)])


BLK = 1024
GRID = N_PAD // BLK


def _norm_from_deg(d0, d1):
    deg = (d0 + d1)[:, 0:1]
    return jnp.where(deg > 0, lax.rsqrt(jnp.maximum(deg, 1.0)), 0.0)


def _scale_body(x_ref, dego0_ref, dego1_ref, o_ref):
    norm_out = _norm_from_deg(dego0_ref[...], dego1_ref[...])
    o_ref[...] = x_ref[...] * norm_out


def _scale_call(x_pad, dego_p):
    return pl.pallas_call(
        _scale_body,
        out_shape=jax.ShapeDtypeStruct((N_PAD, D), jnp.float32),
        grid=(GRID,),
        in_specs=[
            pl.BlockSpec((BLK, D), lambda i: (i, 0)),
            pl.BlockSpec((BLK, D), lambda i: (i, 0)),
            pl.BlockSpec((BLK, D), lambda i: (i + GRID, 0)),
        ],
        out_specs=pl.BlockSpec((BLK, D), lambda i: (i, 0)),
    )(x_pad, dego_p, dego_p)


def _mm_body(a0_ref, a1_ref, degi0_ref, degi1_ref, dego0_ref, dego1_ref,
             w_ref, b_ref, o_ref, *, relu_and_scale):
    norm_in = _norm_from_deg(degi0_ref[...], degi1_ref[...])
    agg = (a0_ref[...] + a1_ref[...]) * norm_in
    out = jnp.dot(agg, w_ref[...], preferred_element_type=jnp.float32)
    out = out + b_ref[...]
    if relu_and_scale:
        out = jnp.maximum(out, 0.0)
        out = out * _norm_from_deg(dego0_ref[...], dego1_ref[...])
    o_ref[...] = out


def _mm_call(agg_p, degi_p, dego_p, w, b, relu_and_scale):
    body = functools.partial(_mm_body, relu_and_scale=relu_and_scale)
    return pl.pallas_call(
        body,
        out_shape=jax.ShapeDtypeStruct((N_PAD, D), jnp.float32),
        grid=(GRID,),
        in_specs=[
            pl.BlockSpec((BLK, D), lambda i: (i, 0)),
            pl.BlockSpec((BLK, D), lambda i: (i + GRID, 0)),
            pl.BlockSpec((BLK, D), lambda i: (i, 0)),
            pl.BlockSpec((BLK, D), lambda i: (i + GRID, 0)),
            pl.BlockSpec((BLK, D), lambda i: (i, 0)),
            pl.BlockSpec((BLK, D), lambda i: (i + GRID, 0)),
            pl.BlockSpec((D, D), lambda i: (0, 0)),
            pl.BlockSpec((1, D), lambda i: (0, 0)),
        ],
        out_specs=pl.BlockSpec((BLK, D), lambda i: (i, 0)),
    )(agg_p, agg_p, degi_p, degi_p, dego_p, dego_p, w, b)


def kernel(x, edge_index, W1, b1, W2, b2):
    src = edge_index[0]
    dst = edge_index[1]
    pad_idx = jnp.full((E_PAD - E_EDGES,), N_NODES, jnp.int32)
    src_p = jnp.concatenate([src, pad_idx])
    dst_p = jnp.concatenate([dst, pad_idx])
    x_pad = jnp.pad(x, ((0, N_PAD - N_NODES), (0, 0)))
    b1r = b1.reshape(1, D)
    b2r = b2.reshape(1, D)

    dego_p = _deg_kernel(src_p)
    degi_p = _deg_kernel(dst_p)
    h1 = _scale_call(x_pad, dego_p)
    agg1 = _spmm_kernel(h1, src_p, dst_p)
    h2 = _mm_call(agg1, degi_p, dego_p, W1, b1r, True)
    agg2 = _spmm_kernel(h2, src_p, dst_p)
    out = _mm_call(agg2, degi_p, dego_p, W2, b2r, False)
    return out[:N_NODES]
